# Initial kernel scaffold; baseline (speedup 1.0000x reference)
import jax
import jax.numpy as jnp
from jax.experimental import pallas as pl


def kernel(x, attr, W1, b1, gamma, beta, W2, b2):
    raise NotImplementedError("write your pallas kernel here")


# speedup vs baseline: 2.5987x; 2.5987x over previous
import jax
import jax.numpy as jnp
from jax.experimental import pallas as pl

_B, _D = 16384, 512
_NIN, _ADIM, _HID, _OC = 128, 16, 64, 128
_IDX = 3
_OS = 128
_R = 1024
_NB = _B // _R
_EPS = 1e-5


def _stage1(x_ref, a_ref, w1a_ref, w1b_ref, b1_ref, h_ref, st_ref):
    i = pl.program_id(0)
    h = (jnp.dot(x_ref[...], w1a_ref[...], preferred_element_type=jnp.float32)
         + jnp.dot(a_ref[...], w1b_ref[...], preferred_element_type=jnp.float32)
         + b1_ref[...])
    h_ref[...] = h
    s = jnp.sum(h, axis=0, keepdims=True)
    s2 = jnp.sum(h * h, axis=0, keepdims=True)
    upd = jnp.concatenate([s, s2, jnp.zeros((6, _HID), jnp.float32)], axis=0)

    @pl.when(i == 0)
    def _init():
        st_ref[...] = jnp.zeros_like(st_ref)

    st_ref[...] += upd


def _stage2(h_ref, st_ref, g_ref, be_ref, w2_ref, b2_ref, o_ref):
    st = st_ref[...]
    mean = st[0:1, :] * (1.0 / _B)
    var = st[1:2, :] * (1.0 / _B) - mean * mean
    scale = jax.lax.rsqrt(var + _EPS) * g_ref[...]
    shift = be_ref[...] - mean * scale
    hn = h_ref[...] * scale + shift
    he = jnp.where(hn > 0, hn, jnp.exp(hn) - 1.0)
    out = jnp.tanh(jnp.dot(he, w2_ref[...], preferred_element_type=jnp.float32)
                   + b2_ref[...])
    o_ref[...] = jnp.zeros((o_ref.shape[0], _D), jnp.float32)
    o_ref[:, _OS:_OS + _OC] = out


def kernel(x, attr, W1, b1, gamma, beta, W2, b2):
    a = attr[:, _IDX, :]
    w1a = W1[:, :_NIN].T
    w1b = W1[:, _NIN:].T
    w2 = W2.T
    h, st = pl.pallas_call(
        _stage1,
        grid=(_NB,),
        in_specs=[
            pl.BlockSpec((_R, _NIN), lambda i: (i, 0)),
            pl.BlockSpec((_R, _ADIM), lambda i: (i, 0)),
            pl.BlockSpec((_NIN, _HID), lambda i: (0, 0)),
            pl.BlockSpec((_ADIM, _HID), lambda i: (0, 0)),
            pl.BlockSpec((1, _HID), lambda i: (0, 0)),
        ],
        out_specs=[
            pl.BlockSpec((_R, _HID), lambda i: (i, 0)),
            pl.BlockSpec((8, _HID), lambda i: (0, 0)),
        ],
        out_shape=[
            jax.ShapeDtypeStruct((_B, _HID), jnp.float32),
            jax.ShapeDtypeStruct((8, _HID), jnp.float32),
        ],
    )(x, a, w1a, w1b, b1.reshape(1, _HID))
    res = pl.pallas_call(
        _stage2,
        grid=(_NB,),
        in_specs=[
            pl.BlockSpec((_R, _HID), lambda i: (i, 0)),
            pl.BlockSpec((8, _HID), lambda i: (0, 0)),
            pl.BlockSpec((1, _HID), lambda i: (0, 0)),
            pl.BlockSpec((1, _HID), lambda i: (0, 0)),
            pl.BlockSpec((_HID, _OC), lambda i: (0, 0)),
            pl.BlockSpec((1, _OC), lambda i: (0, 0)),
        ],
        out_specs=pl.BlockSpec((_R, _D), lambda i: (i, 0)),
        out_shape=jax.ShapeDtypeStruct((_B, _D), jnp.float32),
    )(h, st, gamma.reshape(1, _HID), beta.reshape(1, _HID), w2,
      b2.reshape(1, _OC))
    return res


# speedup vs baseline: 3.1897x; 1.2274x over previous
import jax
import jax.numpy as jnp
from jax.experimental import pallas as pl

_B, _D = 16384, 512
_NIN, _ADIM, _HID, _OC = 128, 16, 64, 128
_IDX = 3
_OS = 128
_R = 2048
_NB = _B // _R
_EPS = 1e-5


def _stage1(x_ref, a_ref, w1a_ref, w1b_ref, b1_ref, h_ref, st_ref):
    i = pl.program_id(0)
    h = (jnp.dot(x_ref[...], w1a_ref[...], preferred_element_type=jnp.float32)
         + jnp.dot(a_ref[...], w1b_ref[...], preferred_element_type=jnp.float32)
         + b1_ref[...])
    h_ref[...] = h
    s = jnp.sum(h, axis=0, keepdims=True)
    s2 = jnp.sum(h * h, axis=0, keepdims=True)
    upd = jnp.concatenate([s, s2, jnp.zeros((6, _HID), jnp.float32)], axis=0)

    @pl.when(i == 0)
    def _init():
        st_ref[...] = jnp.zeros_like(st_ref)

    st_ref[...] += upd


def _stage2(h_ref, st_ref, g_ref, be_ref, w2_ref, b2_ref, o_ref):
    st = st_ref[...]
    mean = st[0:1, :] * (1.0 / _B)
    var = st[1:2, :] * (1.0 / _B) - mean * mean
    scale = jax.lax.rsqrt(var + _EPS) * g_ref[...]
    shift = be_ref[...] - mean * scale
    hn = h_ref[...] * scale + shift
    he = jnp.where(hn > 0, hn, jnp.exp(hn) - 1.0)
    out = jnp.tanh(jnp.dot(he, w2_ref[...], preferred_element_type=jnp.float32)
                   + b2_ref[...])
    o_ref[...] = jnp.zeros((o_ref.shape[0], _D), jnp.float32)
    o_ref[:, _OS:_OS + _OC] = out


def kernel(x, attr, W1, b1, gamma, beta, W2, b2):
    a = attr[:, _IDX, :]
    w1a = W1[:, :_NIN].T
    w1b = W1[:, _NIN:].T
    w2 = W2.T
    h, st = pl.pallas_call(
        _stage1,
        grid=(_NB,),
        in_specs=[
            pl.BlockSpec((_R, _NIN), lambda i: (i, 0)),
            pl.BlockSpec((_R, _ADIM), lambda i: (i, 0)),
            pl.BlockSpec((_NIN, _HID), lambda i: (0, 0)),
            pl.BlockSpec((_ADIM, _HID), lambda i: (0, 0)),
            pl.BlockSpec((1, _HID), lambda i: (0, 0)),
        ],
        out_specs=[
            pl.BlockSpec((_R, _HID), lambda i: (i, 0)),
            pl.BlockSpec((8, _HID), lambda i: (0, 0)),
        ],
        out_shape=[
            jax.ShapeDtypeStruct((_B, _HID), jnp.float32),
            jax.ShapeDtypeStruct((8, _HID), jnp.float32),
        ],
    )(x, a, w1a, w1b, b1.reshape(1, _HID))
    res = pl.pallas_call(
        _stage2,
        grid=(_NB,),
        in_specs=[
            pl.BlockSpec((_R, _HID), lambda i: (i, 0)),
            pl.BlockSpec((8, _HID), lambda i: (0, 0)),
            pl.BlockSpec((1, _HID), lambda i: (0, 0)),
            pl.BlockSpec((1, _HID), lambda i: (0, 0)),
            pl.BlockSpec((_HID, _OC), lambda i: (0, 0)),
            pl.BlockSpec((1, _OC), lambda i: (0, 0)),
        ],
        out_specs=pl.BlockSpec((_R, _D), lambda i: (i, 0)),
        out_shape=jax.ShapeDtypeStruct((_B, _D), jnp.float32),
    )(h, st, gamma.reshape(1, _HID), beta.reshape(1, _HID), w2,
      b2.reshape(1, _OC))
    return res


# speedup vs baseline: 3.4121x; 1.0697x over previous
import jax
import jax.numpy as jnp
from jax.experimental import pallas as pl

_B, _D = 16384, 512
_NIN, _ADIM, _HID, _OC = 128, 16, 64, 128
_IDX = 3
_OS = 128
_R = 4096
_NB = _B // _R
_EPS = 1e-5


def _stage1(x_ref, a_ref, w1a_ref, w1b_ref, b1_ref, h_ref, st_ref):
    i = pl.program_id(0)
    h = (jnp.dot(x_ref[...], w1a_ref[...], preferred_element_type=jnp.float32)
         + jnp.dot(a_ref[...], w1b_ref[...], preferred_element_type=jnp.float32)
         + b1_ref[...])
    h_ref[...] = h
    s = jnp.sum(h, axis=0, keepdims=True)
    s2 = jnp.sum(h * h, axis=0, keepdims=True)
    upd = jnp.concatenate([s, s2, jnp.zeros((6, _HID), jnp.float32)], axis=0)

    @pl.when(i == 0)
    def _init():
        st_ref[...] = jnp.zeros_like(st_ref)

    st_ref[...] += upd


def _stage2(h_ref, st_ref, g_ref, be_ref, w2_ref, b2_ref, o_ref):
    st = st_ref[...]
    mean = st[0:1, :] * (1.0 / _B)
    var = st[1:2, :] * (1.0 / _B) - mean * mean
    scale = jax.lax.rsqrt(var + _EPS) * g_ref[...]
    shift = be_ref[...] - mean * scale
    hn = h_ref[...] * scale + shift
    he = jnp.where(hn > 0, hn, jnp.exp(hn) - 1.0)
    out = jnp.tanh(jnp.dot(he, w2_ref[...], preferred_element_type=jnp.float32)
                   + b2_ref[...])
    o_ref[...] = jnp.zeros((o_ref.shape[0], _D), jnp.float32)
    o_ref[:, _OS:_OS + _OC] = out


def kernel(x, attr, W1, b1, gamma, beta, W2, b2):
    a = attr[:, _IDX, :]
    w1a = W1[:, :_NIN].T
    w1b = W1[:, _NIN:].T
    w2 = W2.T
    h, st = pl.pallas_call(
        _stage1,
        grid=(_NB,),
        in_specs=[
            pl.BlockSpec((_R, _NIN), lambda i: (i, 0)),
            pl.BlockSpec((_R, _ADIM), lambda i: (i, 0)),
            pl.BlockSpec((_NIN, _HID), lambda i: (0, 0)),
            pl.BlockSpec((_ADIM, _HID), lambda i: (0, 0)),
            pl.BlockSpec((1, _HID), lambda i: (0, 0)),
        ],
        out_specs=[
            pl.BlockSpec((_R, _HID), lambda i: (i, 0)),
            pl.BlockSpec((8, _HID), lambda i: (0, 0)),
        ],
        out_shape=[
            jax.ShapeDtypeStruct((_B, _HID), jnp.float32),
            jax.ShapeDtypeStruct((8, _HID), jnp.float32),
        ],
    )(x, a, w1a, w1b, b1.reshape(1, _HID))
    res = pl.pallas_call(
        _stage2,
        grid=(_NB,),
        in_specs=[
            pl.BlockSpec((_R, _HID), lambda i: (i, 0)),
            pl.BlockSpec((8, _HID), lambda i: (0, 0)),
            pl.BlockSpec((1, _HID), lambda i: (0, 0)),
            pl.BlockSpec((1, _HID), lambda i: (0, 0)),
            pl.BlockSpec((_HID, _OC), lambda i: (0, 0)),
            pl.BlockSpec((1, _OC), lambda i: (0, 0)),
        ],
        out_specs=pl.BlockSpec((_R, _D), lambda i: (i, 0)),
        out_shape=jax.ShapeDtypeStruct((_B, _D), jnp.float32),
    )(h, st, gamma.reshape(1, _HID), beta.reshape(1, _HID), w2,
      b2.reshape(1, _OC))
    return res


# speedup vs baseline: 3.4256x; 1.0040x over previous
import jax
import jax.numpy as jnp
from jax.experimental import pallas as pl

_B, _D = 16384, 512
_NIN, _ADIM, _HID, _OC = 128, 16, 64, 128
_IDX = 3
_OS = 128
_R = 8192
_NB = _B // _R
_EPS = 1e-5


def _stage1(x_ref, a_ref, w1a_ref, w1b_ref, b1_ref, h_ref, st_ref):
    i = pl.program_id(0)
    h = (jnp.dot(x_ref[...], w1a_ref[...], preferred_element_type=jnp.float32)
         + jnp.dot(a_ref[...], w1b_ref[...], preferred_element_type=jnp.float32)
         + b1_ref[...])
    h_ref[...] = h
    s = jnp.sum(h, axis=0, keepdims=True)
    s2 = jnp.sum(h * h, axis=0, keepdims=True)
    upd = jnp.concatenate([s, s2, jnp.zeros((6, _HID), jnp.float32)], axis=0)

    @pl.when(i == 0)
    def _init():
        st_ref[...] = jnp.zeros_like(st_ref)

    st_ref[...] += upd


def _stage2(h_ref, st_ref, g_ref, be_ref, w2_ref, b2_ref, o_ref):
    st = st_ref[...]
    mean = st[0:1, :] * (1.0 / _B)
    var = st[1:2, :] * (1.0 / _B) - mean * mean
    scale = jax.lax.rsqrt(var + _EPS) * g_ref[...]
    shift = be_ref[...] - mean * scale
    hn = h_ref[...] * scale + shift
    he = jnp.where(hn > 0, hn, jnp.exp(hn) - 1.0)
    out = jnp.tanh(jnp.dot(he, w2_ref[...], preferred_element_type=jnp.float32)
                   + b2_ref[...])
    o_ref[...] = jnp.zeros((o_ref.shape[0], _D), jnp.float32)
    o_ref[:, _OS:_OS + _OC] = out


def kernel(x, attr, W1, b1, gamma, beta, W2, b2):
    a = attr[:, _IDX, :]
    w1a = W1[:, :_NIN].T
    w1b = W1[:, _NIN:].T
    w2 = W2.T
    h, st = pl.pallas_call(
        _stage1,
        grid=(_NB,),
        in_specs=[
            pl.BlockSpec((_R, _NIN), lambda i: (i, 0)),
            pl.BlockSpec((_R, _ADIM), lambda i: (i, 0)),
            pl.BlockSpec((_NIN, _HID), lambda i: (0, 0)),
            pl.BlockSpec((_ADIM, _HID), lambda i: (0, 0)),
            pl.BlockSpec((1, _HID), lambda i: (0, 0)),
        ],
        out_specs=[
            pl.BlockSpec((_R, _HID), lambda i: (i, 0)),
            pl.BlockSpec((8, _HID), lambda i: (0, 0)),
        ],
        out_shape=[
            jax.ShapeDtypeStruct((_B, _HID), jnp.float32),
            jax.ShapeDtypeStruct((8, _HID), jnp.float32),
        ],
    )(x, a, w1a, w1b, b1.reshape(1, _HID))
    res = pl.pallas_call(
        _stage2,
        grid=(_NB,),
        in_specs=[
            pl.BlockSpec((_R, _HID), lambda i: (i, 0)),
            pl.BlockSpec((8, _HID), lambda i: (0, 0)),
            pl.BlockSpec((1, _HID), lambda i: (0, 0)),
            pl.BlockSpec((1, _HID), lambda i: (0, 0)),
            pl.BlockSpec((_HID, _OC), lambda i: (0, 0)),
            pl.BlockSpec((1, _OC), lambda i: (0, 0)),
        ],
        out_specs=pl.BlockSpec((_R, _D), lambda i: (i, 0)),
        out_shape=jax.ShapeDtypeStruct((_B, _D), jnp.float32),
    )(h, st, gamma.reshape(1, _HID), beta.reshape(1, _HID), w2,
      b2.reshape(1, _OC))
    return res


# speedup vs baseline: 3.8784x; 1.1322x over previous
import jax
import jax.numpy as jnp
from jax.experimental import pallas as pl
from jax.experimental.pallas import tpu as pltpu

_B, _D = 16384, 512
_NIN, _ADIM, _HID, _OC = 128, 16, 64, 128
_IDX = 3
_OS = 128
_R = 2048
_NB = _B // _R
_EPS = 1e-5


def _fused(x_ref, a_ref, w1a_ref, w1b_ref, b1_ref, g_ref, be_ref, w2_ref,
           b2_ref, o_ref, h_ref, sc_ref):
    i = pl.program_id(0)

    @pl.when(i == 0)
    def _stage1():
        h = (jnp.dot(x_ref[...], w1a_ref[...],
                     preferred_element_type=jnp.float32)
             + jnp.dot(a_ref[...], w1b_ref[...],
                       preferred_element_type=jnp.float32)
             + b1_ref[...])
        h_ref[...] = h
        mean = jnp.sum(h, axis=0, keepdims=True) * (1.0 / _B)
        var = jnp.sum(h * h, axis=0, keepdims=True) * (1.0 / _B) - mean * mean
        scale = jax.lax.rsqrt(var + _EPS) * g_ref[...]
        shift = be_ref[...] - mean * scale
        sc_ref[0:1, :] = scale
        sc_ref[1:2, :] = shift

    scale = sc_ref[0:1, :]
    shift = sc_ref[1:2, :]
    hn = h_ref[pl.ds(i * _R, _R), :] * scale + shift
    he = jnp.where(hn > 0, hn, jnp.exp(hn) - 1.0)
    out = jnp.tanh(jnp.dot(he, w2_ref[...], preferred_element_type=jnp.float32)
                   + b2_ref[...])
    o_ref[...] = jnp.zeros((_R, _D), jnp.float32)
    o_ref[:, _OS:_OS + _OC] = out


def kernel(x, attr, W1, b1, gamma, beta, W2, b2):
    a = attr[:, _IDX, :]
    w1a = W1[:, :_NIN].T
    w1b = W1[:, _NIN:].T
    w2 = W2.T
    const = lambda i: (0, 0)
    res = pl.pallas_call(
        _fused,
        grid=(_NB,),
        in_specs=[
            pl.BlockSpec((_B, _NIN), const),
            pl.BlockSpec((_B, _ADIM), const),
            pl.BlockSpec((_NIN, _HID), const),
            pl.BlockSpec((_ADIM, _HID), const),
            pl.BlockSpec((1, _HID), const),
            pl.BlockSpec((1, _HID), const),
            pl.BlockSpec((1, _HID), const),
            pl.BlockSpec((_HID, _OC), const),
            pl.BlockSpec((1, _OC), const),
        ],
        out_specs=pl.BlockSpec((_R, _D), lambda i: (i, 0)),
        out_shape=jax.ShapeDtypeStruct((_B, _D), jnp.float32),
        scratch_shapes=[
            pltpu.VMEM((_B, _HID), jnp.float32),
            pltpu.VMEM((8, _HID), jnp.float32),
        ],
    )(x, a, w1a, w1b, b1.reshape(1, _HID), gamma.reshape(1, _HID),
      beta.reshape(1, _HID), w2, b2.reshape(1, _OC))
    return res
